# baseline (speedup 1.0000x reference)
import functools

import jax
import jax.numpy as jnp
import numpy as np
from jax import lax
from jax.experimental import pallas as pl
from jax.experimental.pallas import tpu as pltpu
from jax.experimental.pallas import tpu_sc as plsc

B = 16
N0 = 1024
EPS = 1e-5
_SCALE = float(1.0 / np.sqrt(1.0 + EPS))

_NW = 32


@functools.lru_cache(maxsize=1)
def _sc_mesh():
    return plsc.VectorSubcoreMesh(core_axis_name="c", subcore_axis_name="s")


@functools.lru_cache(maxsize=1)
def _sc_params():
    return pltpu.CompilerParams(needs_layout_passes=False,
                                use_tc_tiling_on_sc=False)


def _fold(w, g):
    return w * (g * _SCALE)[:, None]


def _fps_body(x_ref, y_ref, z_ref, ocx_ref, ocy_ref, ocz_ref, *, npoint, n):
    X = x_ref[...]
    Y = y_ref[...]
    Z = z_ref[...]
    iota = lax.broadcasted_iota(jnp.int32, (B, n), 1)
    piota = lax.broadcasted_iota(jnp.int32, (B, npoint), 1)

    def body(i, st):
        dists, far, ocx, ocy, ocz = st
        oh = jnp.where(iota == far, 1.0, 0.0)
        cx = jnp.sum(X * oh, axis=1, keepdims=True)
        cy = jnp.sum(Y * oh, axis=1, keepdims=True)
        cz = jnp.sum(Z * oh, axis=1, keepdims=True)
        sel = piota == i
        ocx = jnp.where(sel, cx, ocx)
        ocy = jnp.where(sel, cy, ocy)
        ocz = jnp.where(sel, cz, ocz)
        dx = X - cx
        dy = Y - cy
        dz = Z - cz
        d = (dx * dx + dy * dy) + dz * dz
        dists = jnp.minimum(dists, d)
        m = jnp.max(dists, axis=1, keepdims=True)
        far = jnp.min(jnp.where(dists == m, iota, n), axis=1, keepdims=True)
        far = far.astype(jnp.int32)
        return dists, far, ocx, ocy, ocz

    dists = jnp.full((B, n), 1e10, jnp.float32)
    far = jnp.zeros((B, 1), jnp.int32)
    zc = jnp.zeros((B, npoint), jnp.float32)
    _, _, ocx, ocy, ocz = lax.fori_loop(0, npoint, body, (dists, far, zc, zc, zc))
    ocx_ref[...] = ocx
    ocy_ref[...] = ocy
    ocz_ref[...] = ocz


def _fps(X, Y, Z, npoint):
    n = X.shape[1]
    out = jax.ShapeDtypeStruct((B, npoint), jnp.float32)
    return pl.pallas_call(
        functools.partial(_fps_body, npoint=npoint, n=n),
        out_shape=(out, out, out),
    )(X, Y, Z)


def _select_body(refs, *, S, n, r2, rpw, nsample, coords):
    if coords:
        (cx_hbm, cy_hbm, cz_hbm, x_hbm, y_hbm, z_hbm,
         ox_hbm, oy_hbm, oz_hbm,
         cxv, cyv, czv, xv, yv, zv,
         xb0, yb0, zb0, xb1, yb1, zb1, xb2, yb2, zb2, xb3, yb3, zb3,
         oxv, oyv, ozv) = refs
        xbs = (xb0, xb1, xb2, xb3)
        ybs = (yb0, yb1, yb2, yb3)
        zbs = (zb0, zb1, zb2, zb3)
    else:
        (cx_hbm, cy_hbm, cz_hbm, x_hbm, y_hbm, z_hbm,
         oi_hbm, cxv, cyv, czv, xv, yv, zv, bufv, oiv) = refs
    w = lax.axis_index("s") * 2 + lax.axis_index("c")
    base_row = w * rpw
    b = base_row // S
    base_n = b * n
    pltpu.sync_copy(cx_hbm.at[pl.ds(base_row, rpw)], cxv)
    pltpu.sync_copy(cy_hbm.at[pl.ds(base_row, rpw)], cyv)
    pltpu.sync_copy(cz_hbm.at[pl.ds(base_row, rpw)], czv)
    pltpu.sync_copy(x_hbm.at[pl.ds(base_n, n)], xv)
    pltpu.sync_copy(y_hbm.at[pl.ds(base_n, n)], yv)
    pltpu.sync_copy(z_hbm.at[pl.ds(base_n, n)], zv)
    io16 = lax.iota(jnp.int32, 16)

    def quad_fn(q, _):
        r0 = q * 4
        cs = []
        for i in range(4):
            rvec = jnp.broadcast_to(r0 + i, (16,))
            cs.append((plsc.load_gather(cxv, [rvec]),
                       plsc.load_gather(cyv, [rvec]),
                       plsc.load_gather(czv, [rvec])))
        x0 = jnp.broadcast_to(xv[pl.ds(0, 16)][0], (16,))
        y0 = jnp.broadcast_to(yv[pl.ds(0, 16)][0], (16,))
        z0 = jnp.broadcast_to(zv[pl.ds(0, 16)][0], (16,))
        for i in range(4):
            xbs[i][pl.ds(0, 16)] = x0 - cs[i][0]
            ybs[i][pl.ds(0, 16)] = y0 - cs[i][1]
            zbs[i][pl.ds(0, 16)] = z0 - cs[i][2]

        def chunk_fn(j, cnts):
            off = j * 16
            xc = xv[pl.ds(off, 16)]
            yc = yv[pl.ds(off, 16)]
            zc = zv[pl.ds(off, 16)]
            out = []
            for i in range(4):
                dx = xc - cs[i][0]
                dy = yc - cs[i][1]
                dz = zc - cs[i][2]
                d = (dx * dx + dy * dy) + dz * dz
                m = d <= r2
                plsc.store_compressed(xbs[i].at[pl.ds(cnts[i], 16)], dx, mask=m)
                plsc.store_compressed(ybs[i].at[pl.ds(cnts[i], 16)], dy, mask=m)
                plsc.store_compressed(zbs[i].at[pl.ds(cnts[i], 16)], dz, mask=m)
                out.append(cnts[i] + plsc.all_reduce_population_count(m)[0])
            return tuple(out)

        cnts = lax.fori_loop(0, n // 16, chunk_fn,
                             (jnp.int32(0),) * 4, unroll=2)
        for i in range(4):
            r = r0 + i
            cnt = cnts[i]
            fx = jnp.broadcast_to(xbs[i][pl.ds(0, 16)][0], (16,))
            fy = jnp.broadcast_to(ybs[i][pl.ds(0, 16)][0], (16,))
            fz = jnp.broadcast_to(zbs[i][pl.ds(0, 16)][0], (16,))
            for kc in range(nsample // 16):
                xbs[i][pl.ds(cnt + kc * 16, 16)] = fx
                ybs[i][pl.ds(cnt + kc * 16, 16)] = fy
                zbs[i][pl.ds(cnt + kc * 16, 16)] = fz
            tile = r // 128
            s_in = r - tile * 128
            obase = tile * (128 * nsample) + s_in
            for kc in range(nsample // 16):
                oidx = (io16 + kc * 16) * 128 + obase
                plsc.store_scatter(oxv, [oidx], xbs[i][pl.ds(kc * 16, 16)])
                plsc.store_scatter(oyv, [oidx], ybs[i][pl.ds(kc * 16, 16)])
                plsc.store_scatter(ozv, [oidx], zbs[i][pl.ds(kc * 16, 16)])
        return 0

    def row_fn(r, _):
        rvec = jnp.broadcast_to(r, (16,))
        cxr = plsc.load_gather(cxv, [rvec])
        cyr = plsc.load_gather(cyv, [rvec])
        czr = plsc.load_gather(czv, [rvec])
        bufv[pl.ds(0, 16)] = jnp.broadcast_to(base_n, (16,))

        def chunk_fn(j, cnt):
            off = j * 16
            xc = xv[pl.ds(off, 16)]
            yc = yv[pl.ds(off, 16)]
            zc = zv[pl.ds(off, 16)]
            dx = xc - cxr
            dy = yc - cyr
            dz = zc - czr
            d = (dx * dx + dy * dy) + dz * dz
            m = d <= r2
            gidx = io16 + (off + base_n)
            plsc.store_compressed(bufv.at[pl.ds(cnt, 16)], gidx, mask=m)
            return cnt + plsc.all_reduce_population_count(m)[0]

        cnt = lax.fori_loop(0, n // 16, chunk_fn, jnp.int32(0), unroll=4)
        fill = jnp.broadcast_to(bufv[pl.ds(0, 16)][0], (16,))
        for kc in range(nsample // 16):
            bufv[pl.ds(cnt + kc * 16, 16)] = fill
        for kc in range(nsample // 16):
            oiv[pl.ds(r * nsample + kc * 16, 16)] = bufv[pl.ds(kc * 16, 16)]
        return 0

    if coords:
        lax.fori_loop(0, rpw // 4, quad_fn, 0)
    else:
        lax.fori_loop(0, rpw, row_fn, 0)
    sl = pl.ds(base_row * nsample, rpw * nsample)
    if coords:
        pltpu.sync_copy(oxv, ox_hbm.at[sl])
        pltpu.sync_copy(oyv, oy_hbm.at[sl])
        pltpu.sync_copy(ozv, oz_hbm.at[sl])
    else:
        pltpu.sync_copy(oiv, oi_hbm.at[sl])


def _select(cx, cy, cz, X, Y, Z, r2, nsample=64, coords=False):
    S = cx.shape[1]
    n = X.shape[1]
    rows = B * S
    rpw = rows // _NW
    i_out = jax.ShapeDtypeStruct((rows * nsample,), jnp.int32)
    f_out = jax.ShapeDtypeStruct((rows * nsample,), jnp.float32)
    scratch = [
        pltpu.VMEM((rpw,), jnp.float32),
        pltpu.VMEM((rpw,), jnp.float32),
        pltpu.VMEM((rpw,), jnp.float32),
        pltpu.VMEM((n,), jnp.float32),
        pltpu.VMEM((n,), jnp.float32),
        pltpu.VMEM((n,), jnp.float32),
    ]
    if coords:
        scratch += [pltpu.VMEM((n + 64,), jnp.float32)] * 12
        scratch += [pltpu.VMEM((rpw * nsample,), jnp.float32)] * 3
    else:
        scratch += [pltpu.VMEM((n + 64,), jnp.int32),
                    pltpu.VMEM((rpw * nsample,), jnp.int32)]

    def body(*refs):
        _select_body(refs, S=S, n=n, r2=r2, rpw=rpw, nsample=nsample,
                     coords=coords)

    kfn = pl.kernel(
        out_type=(f_out, f_out, f_out) if coords else i_out,
        mesh=_sc_mesh(),
        compiler_params=_sc_params(),
        scratch_types=scratch,
    )(body)
    out = kfn(cx.reshape(-1), cy.reshape(-1), cz.reshape(-1),
              X.reshape(-1), Y.reshape(-1), Z.reshape(-1))
    if coords:
        return out
    return out.reshape(rows, nsample)


def _gather_body(tab_hbm, idx_hbm, out_hbm, idxv, rows0, rows1,
                 sg0, sg1, so0, so1, *, tpw, chunk):
    w = lax.axis_index("s") * 2 + lax.axis_index("c")
    base = w * tpw
    nc = tpw // chunk
    rows = (rows0, rows1)
    sg = (sg0, sg1)
    so = (so0, so1)
    pltpu.sync_copy(idx_hbm.at[pl.ds(base, tpw)], idxv)
    gathers = [None, None]
    outs = [None, None]
    gathers[0] = pltpu.async_copy(
        tab_hbm.at[idxv.at[pl.ds(0, chunk)]], rows[0], sg[0])
    for c in range(nc):
        bi = c & 1
        gathers[bi].wait()
        if c + 1 < nc:
            ni = (c + 1) & 1
            if outs[ni] is not None:
                outs[ni].wait()
                outs[ni] = None
            gathers[ni] = pltpu.async_copy(
                tab_hbm.at[idxv.at[pl.ds((c + 1) * chunk, chunk)]],
                rows[ni], sg[ni])
        outs[bi] = pltpu.async_copy(
            rows[bi], out_hbm.at[pl.ds(base + c * chunk, chunk)], so[bi])
    for o in outs:
        if o is not None:
            o.wait()


def _gather(tab, idx, chunk):
    tot = idx.shape[0]
    d = tab.shape[1]
    tpw = tot // _NW
    kfn = pl.kernel(
        out_type=jax.ShapeDtypeStruct((tot, d), tab.dtype),
        mesh=_sc_mesh(),
        compiler_params=_sc_params(),
        scratch_types=[
            pltpu.VMEM((tpw,), jnp.int32),
            pltpu.VMEM((chunk, d), tab.dtype),
            pltpu.VMEM((chunk, d), tab.dtype),
            pltpu.SemaphoreType.DMA,
            pltpu.SemaphoreType.DMA,
            pltpu.SemaphoreType.DMA,
            pltpu.SemaphoreType.DMA,
        ],
    )(functools.partial(_gather_body, tpw=tpw, chunk=chunk))
    return kfn(tab, idx)


def _mlp1_body(xg_ref, yg_ref, zg_ref, ct_ref, w1_ref, b1_ref, w2_ref,
               b2_ref, w3_ref, b3_ref, w1a2_ref, w1b2_ref, u2_ref,
               *, st, ns):
    xg = xg_ref[0, 0]
    yg = yg_ref[0, 0]
    zg = zg_ref[0, 0]
    w1 = w1_ref[...]
    z1 = w1[:, 0:1] * xg + w1[:, 1:2] * yg + w1[:, 2:3] * zg + b1_ref[...]
    z1 = jnp.maximum(z1, 0.0)
    z2 = jnp.dot(w2_ref[...], z1, preferred_element_type=jnp.float32)
    z2 = jnp.maximum(z2 + b2_ref[...], 0.0)
    z3 = jnp.dot(w3_ref[...], z2, preferred_element_type=jnp.float32)
    z3 = jnp.maximum(z3 + b3_ref[...], 0.0)
    m = z3
    width = st * ns
    while width > st:
        width //= 2
        m = jnp.maximum(m[:, :width], m[:, width:2 * width])
    feat = m
    u2 = jnp.dot(w1a2_ref[...], ct_ref[0, 0], preferred_element_type=jnp.float32)
    u2 = u2 + jnp.dot(w1b2_ref[...], feat, preferred_element_type=jnp.float32)
    u2_ref[0, 0] = u2


def _mlp1(Xg, Yg, Zg, CT, w1, b1, w2, b2, w3, b3, w1a2, w1b2,
          st=128, ns=64):
    S = CT.shape[-1]
    nt = S // st
    o_next = w1a2.shape[0]
    Xr = Xg.reshape(B, nt, 1, st * ns)
    Yr = Yg.reshape(B, nt, 1, st * ns)
    Zr = Zg.reshape(B, nt, 1, st * ns)
    CTr = CT.reshape(B, 16, nt, st).transpose(0, 2, 1, 3)
    grid = (B, nt)
    def wspec(shape):
        return pl.BlockSpec(shape, lambda b_, t: tuple(0 for _ in shape))
    gspec = pl.BlockSpec((1, 1, 1, st * ns), lambda b_, t: (b_, t, 0, 0))
    out = pl.pallas_call(
        functools.partial(_mlp1_body, st=st, ns=ns),
        grid=grid,
        in_specs=[
            gspec, gspec, gspec,
            pl.BlockSpec((1, 1, 16, st), lambda b_, t: (b_, t, 0, 0)),
            wspec(w1.shape),
            wspec(b1.shape),
            wspec(w2.shape),
            wspec(b2.shape),
            wspec(w3.shape),
            wspec(b3.shape),
            wspec(w1a2.shape),
            wspec(w1b2.shape),
        ],
        out_specs=pl.BlockSpec((1, 1, o_next, st), lambda b_, t: (b_, t, 0, 0)),
        out_shape=jax.ShapeDtypeStruct((B, nt, o_next, st), jnp.float32),
    )(Xr, Yr, Zr, CTr, w1, b1, w2, b2, w3, b3, w1a2, w1b2)
    return out.transpose(0, 1, 3, 2).reshape(B * S, o_next)


def _tail_body(h_ref, w2t_ref, b2_ref, w3t_ref, b3_ref, fw1t_ref, fb1_ref,
               fw2t_ref, fb2_ref, fw3t_ref, fb3_ref, out_ref, *, npts):
    H = h_ref[...]
    z2 = jnp.dot(H, w2t_ref[...], preferred_element_type=jnp.float32)
    z2 = jnp.maximum(z2 + b2_ref[...], 0.0)
    z3 = jnp.dot(z2, w3t_ref[...], preferred_element_type=jnp.float32)
    z3 = jnp.maximum(z3 + b3_ref[...], 0.0)
    g = jnp.max(z3.reshape(B, npts, z3.shape[1]), axis=1)
    y = jnp.dot(g, fw1t_ref[...], preferred_element_type=jnp.float32)
    y = jnp.maximum(y + fb1_ref[...], 0.0)
    y = jnp.dot(y, fw2t_ref[...], preferred_element_type=jnp.float32)
    y = jnp.maximum(y + fb2_ref[...], 0.0)
    y = jnp.dot(y, fw3t_ref[...], preferred_element_type=jnp.float32)
    out_ref[...] = y + fb3_ref[...]


def _tail(H, w2t, b2, w3t, b3, fw1t, fb1, fw2t, fb2, fw3t, fb3):
    npts = H.shape[1]
    return pl.pallas_call(
        functools.partial(_tail_body, npts=npts),
        out_shape=jax.ShapeDtypeStruct((B, fw3t.shape[1]), jnp.float32),
    )(H.reshape(B * npts, H.shape[-1]), w2t, b2, w3t, b3,
      fw1t, fb1, fw2t, fb2, fw3t, fb3)


def _mlp2_body(g_ref, c_ref, w1a2t_ref, b1_ref, w2t_ref, b2_ref, w3t_ref,
               b3_ref, w1a3t_ref, w1b3t_ref, b1_3_ref, h_ref, *, st, ns):
    G = g_ref[0]
    C = c_ref[0]
    v = jnp.dot(C, w1a2t_ref[...], preferred_element_type=jnp.float32)
    c_in = G.shape[1]
    z1 = G.reshape(st, ns, c_in) - v[:, None, :] + b1_ref[...]
    z1 = jnp.maximum(z1, 0.0).reshape(st * ns, c_in)
    z2 = jnp.dot(z1, w2t_ref[...], preferred_element_type=jnp.float32)
    z2 = jnp.maximum(z2 + b2_ref[...], 0.0)
    z3 = jnp.dot(z2, w3t_ref[...], preferred_element_type=jnp.float32)
    o3 = z3.shape[1]
    z3 = jnp.maximum(z3 + b3_ref[...], 0.0)
    feat = jnp.max(z3.reshape(st, ns, o3), axis=1)
    h1 = jnp.dot(C, w1a3t_ref[...], preferred_element_type=jnp.float32)
    h1 = h1 + jnp.dot(feat, w1b3t_ref[...], preferred_element_type=jnp.float32)
    h_ref[0] = jnp.maximum(h1 + b1_3_ref[...], 0.0)


def _mlp2(G, C, w1a2t, b1, w2t, b2, w3t, b3, w1a3t, w1b3t, b1_3, ns=64):
    nb = C.shape[0]
    st = C.shape[1]
    o_out = w1a3t.shape[1]
    Gr = G.reshape(nb, st * ns, G.shape[-1])
    grid = (nb,)
    def wspec(shape):
        return pl.BlockSpec(shape, lambda b_: tuple(0 for _ in shape))
    return pl.pallas_call(
        functools.partial(_mlp2_body, st=st, ns=ns),
        grid=grid,
        in_specs=[
            pl.BlockSpec((1, st * ns, G.shape[-1]), lambda b_: (b_, 0, 0)),
            pl.BlockSpec((1, st, C.shape[-1]), lambda b_: (b_, 0, 0)),
            wspec(w1a2t.shape),
            wspec(b1.shape),
            wspec(w2t.shape),
            wspec(b2.shape),
            wspec(w3t.shape),
            wspec(b3.shape),
            wspec(w1a3t.shape),
            wspec(w1b3t.shape),
            wspec(b1_3.shape),
        ],
        out_specs=pl.BlockSpec((1, st, o_out), lambda b_: (b_, 0, 0)),
        out_shape=jax.ShapeDtypeStruct((nb, st, o_out), jnp.float32),
    )(Gr, C, w1a2t, b1, w2t, b2, w3t, b3, w1a3t, w1b3t, b1_3)


def _padT(w, rows=16):
    wt = w.T
    return jnp.pad(wt, ((0, rows - wt.shape[0]), (0, 0)))


def kernel(pointcloud, params):
    X = pointcloud[..., 0]
    Y = pointcloud[..., 1]
    Z = pointcloud[..., 2]
    p1, p2, p3, fc = params["sa1"], params["sa2"], params["sa3"], params["fc"]

    w1p = jnp.pad(_fold(p1["w"][0], p1["gamma"][0]),
                  ((0, 0), (0, 13)))
    b1c = p1["beta"][0].reshape(-1, 1)
    w2r = _fold(p1["w"][1], p1["gamma"][1])
    b2c = p1["beta"][1].reshape(-1, 1)
    w3r = _fold(p1["w"][2], p1["gamma"][2])
    b3c = p1["beta"][2].reshape(-1, 1)
    w1_2 = _fold(p2["w"][0], p2["gamma"][0])
    w1a2p = jnp.pad(w1_2[:, :3], ((0, 0), (0, 13)))
    w1a2t = _padT(w1_2[:, :3])
    w1b2r = w1_2[:, 3:]
    b1_2 = p2["beta"][0].reshape(1, -1)
    w2_2t = _fold(p2["w"][1], p2["gamma"][1]).T
    b2_2 = p2["beta"][1].reshape(1, -1)
    w3_2t = _fold(p2["w"][2], p2["gamma"][2]).T
    b3_2 = p2["beta"][2].reshape(1, -1)
    w1_3 = _fold(p3["w"][0], p3["gamma"][0])
    w1a3t = _padT(w1_3[:, :3])
    w1b3t = w1_3[:, 3:].T
    b1_3 = p3["beta"][0].reshape(1, -1)
    w2_3t = _fold(p3["w"][1], p3["gamma"][1]).T
    b2_3 = p3["beta"][1].reshape(1, -1)
    w3_3t = _fold(p3["w"][2], p3["gamma"][2]).T
    b3_3 = p3["beta"][2].reshape(1, -1)
    fw1t = _fold(fc["w1"], fc["g1"]).T
    fw2t = _fold(fc["w2"], fc["g2"]).T
    fw3t = fc["w3"].T

    cx1, cy1, cz1 = _fps(X, Y, Z, 512)
    Xg, Yg, Zg = _select(cx1, cy1, cz1, X, Y, Z, 0.2 * 0.2, coords=True)
    CT1 = jnp.pad(jnp.stack([cx1, cy1, cz1], axis=1), ((0, 0), (0, 13), (0, 0)))
    u2tab = _mlp1(Xg, Yg, Zg, CT1, w1p, b1c, w2r, b2c, w3r, b3c, w1a2p, w1b2r)

    cx2, cy2, cz2 = _fps(cx1, cy1, cz1, 128)
    idx2 = _select(cx2, cy2, cz2, cx1, cy1, cz1, 0.4 * 0.4)
    C2 = jnp.pad(jnp.stack([cx2, cy2, cz2], axis=-1), ((0, 0), (0, 0), (0, 13)))
    idx2f = idx2.reshape(-1)
    half = idx2f.shape[0] // 2
    G2a = _gather(u2tab, idx2f[:half], chunk=256)
    G2b = _gather(u2tab, idx2f[half:], chunk=256)
    Ha = _mlp2(G2a, C2[:B // 2], w1a2t, b1_2, w2_2t, b2_2, w3_2t, b3_2,
               w1a3t, w1b3t, b1_3)
    Hb = _mlp2(G2b, C2[B // 2:], w1a2t, b1_2, w2_2t, b2_2, w3_2t, b3_2,
               w1a3t, w1b3t, b1_3)
    H = jnp.concatenate([Ha, Hb], axis=0)

    return _tail(H, w2_3t, b2_3, w3_3t, b3_3,
                 fw1t, fc["b1"].reshape(1, -1), fw2t, fc["b2"].reshape(1, -1),
                 fw3t, fc["bias3"].reshape(1, -1))


# speedup vs baseline: 1.0046x; 1.0046x over previous
import functools

import jax
import jax.numpy as jnp
import numpy as np
from jax import lax
from jax.experimental import pallas as pl
from jax.experimental.pallas import tpu as pltpu
from jax.experimental.pallas import tpu_sc as plsc

B = 16
N0 = 1024
EPS = 1e-5
_SCALE = float(1.0 / np.sqrt(1.0 + EPS))

_NW = 32


@functools.lru_cache(maxsize=1)
def _sc_mesh():
    return plsc.VectorSubcoreMesh(core_axis_name="c", subcore_axis_name="s")


@functools.lru_cache(maxsize=1)
def _sc_params():
    return pltpu.CompilerParams(needs_layout_passes=False,
                                use_tc_tiling_on_sc=False)


def _fold(w, g):
    return w * (g * _SCALE)[:, None]


def _fps_body(x_ref, y_ref, z_ref, ocx_ref, ocy_ref, ocz_ref, *, npoint, n):
    X = x_ref[...]
    Y = y_ref[...]
    Z = z_ref[...]
    iota = lax.broadcasted_iota(jnp.int32, (B, n), 1)
    piota = lax.broadcasted_iota(jnp.int32, (B, npoint), 1)

    def body(i, st):
        dists, far, ocx, ocy, ocz = st
        oh = jnp.where(iota == far, 1.0, 0.0)
        cx = jnp.sum(X * oh, axis=1, keepdims=True)
        cy = jnp.sum(Y * oh, axis=1, keepdims=True)
        cz = jnp.sum(Z * oh, axis=1, keepdims=True)
        sel = piota == i
        ocx = jnp.where(sel, cx, ocx)
        ocy = jnp.where(sel, cy, ocy)
        ocz = jnp.where(sel, cz, ocz)
        dx = X - cx
        dy = Y - cy
        dz = Z - cz
        d = (dx * dx + dy * dy) + dz * dz
        dists = jnp.minimum(dists, d)
        m = jnp.max(dists, axis=1, keepdims=True)
        far = jnp.min(jnp.where(dists == m, iota, n), axis=1, keepdims=True)
        far = far.astype(jnp.int32)
        return dists, far, ocx, ocy, ocz

    dists = jnp.full((B, n), 1e10, jnp.float32)
    far = jnp.zeros((B, 1), jnp.int32)
    zc = jnp.zeros((B, npoint), jnp.float32)
    _, _, ocx, ocy, ocz = lax.fori_loop(0, npoint, body, (dists, far, zc, zc, zc))
    ocx_ref[...] = ocx
    ocy_ref[...] = ocy
    ocz_ref[...] = ocz


def _fps(X, Y, Z, npoint):
    n = X.shape[1]
    out = jax.ShapeDtypeStruct((B, npoint), jnp.float32)
    return pl.pallas_call(
        functools.partial(_fps_body, npoint=npoint, n=n),
        out_shape=(out, out, out),
    )(X, Y, Z)


def _select_body(refs, *, S, n, r2, rpw, nsample, coords):
    if coords:
        (cx_hbm, cy_hbm, cz_hbm, x_hbm, y_hbm, z_hbm,
         ox_hbm, oy_hbm, oz_hbm,
         cxv, cyv, czv, xv, yv, zv,
         xb0, yb0, zb0, xb1, yb1, zb1, xb2, yb2, zb2, xb3, yb3, zb3,
         oxv, oyv, ozv) = refs
        xbs = (xb0, xb1, xb2, xb3)
        ybs = (yb0, yb1, yb2, yb3)
        zbs = (zb0, zb1, zb2, zb3)
    else:
        (cx_hbm, cy_hbm, cz_hbm, x_hbm, y_hbm, z_hbm,
         oi_hbm, cxv, cyv, czv, xv, yv, zv, bufv, oiv) = refs
    w = lax.axis_index("s") * 2 + lax.axis_index("c")
    base_row = w * rpw
    b = base_row // S
    base_n = b * n
    pltpu.sync_copy(cx_hbm.at[pl.ds(base_row, rpw)], cxv)
    pltpu.sync_copy(cy_hbm.at[pl.ds(base_row, rpw)], cyv)
    pltpu.sync_copy(cz_hbm.at[pl.ds(base_row, rpw)], czv)
    pltpu.sync_copy(x_hbm.at[pl.ds(base_n, n)], xv)
    pltpu.sync_copy(y_hbm.at[pl.ds(base_n, n)], yv)
    pltpu.sync_copy(z_hbm.at[pl.ds(base_n, n)], zv)
    io16 = lax.iota(jnp.int32, 16)

    def quad_fn(q, _):
        r0 = q * 4
        cs = []
        for i in range(4):
            rvec = jnp.broadcast_to(r0 + i, (16,))
            cs.append((plsc.load_gather(cxv, [rvec]),
                       plsc.load_gather(cyv, [rvec]),
                       plsc.load_gather(czv, [rvec])))
        x0 = jnp.broadcast_to(xv[pl.ds(0, 16)][0], (16,))
        y0 = jnp.broadcast_to(yv[pl.ds(0, 16)][0], (16,))
        z0 = jnp.broadcast_to(zv[pl.ds(0, 16)][0], (16,))
        for i in range(4):
            xbs[i][pl.ds(0, 16)] = x0 - cs[i][0]
            ybs[i][pl.ds(0, 16)] = y0 - cs[i][1]
            zbs[i][pl.ds(0, 16)] = z0 - cs[i][2]

        def chunk_fn(j, cnts):
            off = j * 16
            xc = xv[pl.ds(off, 16)]
            yc = yv[pl.ds(off, 16)]
            zc = zv[pl.ds(off, 16)]
            out = []
            for i in range(4):
                dx = xc - cs[i][0]
                dy = yc - cs[i][1]
                dz = zc - cs[i][2]
                d = (dx * dx + dy * dy) + dz * dz
                m = d <= r2
                plsc.store_compressed(xbs[i].at[pl.ds(cnts[i], 16)], dx, mask=m)
                plsc.store_compressed(ybs[i].at[pl.ds(cnts[i], 16)], dy, mask=m)
                plsc.store_compressed(zbs[i].at[pl.ds(cnts[i], 16)], dz, mask=m)
                out.append(cnts[i] + plsc.all_reduce_population_count(m)[0])
            return tuple(out)

        cnts = lax.fori_loop(0, n // 16, chunk_fn,
                             (jnp.int32(0),) * 4, unroll=2)
        for i in range(4):
            r = r0 + i
            cnt = cnts[i]
            fx = jnp.broadcast_to(xbs[i][pl.ds(0, 16)][0], (16,))
            fy = jnp.broadcast_to(ybs[i][pl.ds(0, 16)][0], (16,))
            fz = jnp.broadcast_to(zbs[i][pl.ds(0, 16)][0], (16,))
            for kc in range(nsample // 16):
                xbs[i][pl.ds(cnt + kc * 16, 16)] = fx
                ybs[i][pl.ds(cnt + kc * 16, 16)] = fy
                zbs[i][pl.ds(cnt + kc * 16, 16)] = fz
            tile = r // 128
            s_in = r - tile * 128
            obase = tile * (128 * nsample) + s_in
            for kc in range(nsample // 16):
                oidx = (io16 + kc * 16) * 128 + obase
                plsc.store_scatter(oxv, [oidx], xbs[i][pl.ds(kc * 16, 16)])
                plsc.store_scatter(oyv, [oidx], ybs[i][pl.ds(kc * 16, 16)])
                plsc.store_scatter(ozv, [oidx], zbs[i][pl.ds(kc * 16, 16)])
        return 0

    def row_fn(r, _):
        rvec = jnp.broadcast_to(r, (16,))
        cxr = plsc.load_gather(cxv, [rvec])
        cyr = plsc.load_gather(cyv, [rvec])
        czr = plsc.load_gather(czv, [rvec])
        bufv[pl.ds(0, 16)] = jnp.broadcast_to(base_n, (16,))

        def chunk_fn(j, cnt):
            off = j * 16
            xc = xv[pl.ds(off, 16)]
            yc = yv[pl.ds(off, 16)]
            zc = zv[pl.ds(off, 16)]
            dx = xc - cxr
            dy = yc - cyr
            dz = zc - czr
            d = (dx * dx + dy * dy) + dz * dz
            m = d <= r2
            gidx = io16 + (off + base_n)
            plsc.store_compressed(bufv.at[pl.ds(cnt, 16)], gidx, mask=m)
            return cnt + plsc.all_reduce_population_count(m)[0]

        cnt = lax.fori_loop(0, n // 16, chunk_fn, jnp.int32(0), unroll=4)
        fill = jnp.broadcast_to(bufv[pl.ds(0, 16)][0], (16,))
        for kc in range(nsample // 16):
            bufv[pl.ds(cnt + kc * 16, 16)] = fill
        for kc in range(nsample // 16):
            oiv[pl.ds(r * nsample + kc * 16, 16)] = bufv[pl.ds(kc * 16, 16)]
        return 0

    if coords:
        lax.fori_loop(0, rpw // 4, quad_fn, 0)
    else:
        lax.fori_loop(0, rpw, row_fn, 0)
    sl = pl.ds(base_row * nsample, rpw * nsample)
    if coords:
        pltpu.sync_copy(oxv, ox_hbm.at[sl])
        pltpu.sync_copy(oyv, oy_hbm.at[sl])
        pltpu.sync_copy(ozv, oz_hbm.at[sl])
    else:
        pltpu.sync_copy(oiv, oi_hbm.at[sl])


def _select(cx, cy, cz, X, Y, Z, r2, nsample=64, coords=False):
    S = cx.shape[1]
    n = X.shape[1]
    rows = B * S
    rpw = rows // _NW
    i_out = jax.ShapeDtypeStruct((rows * nsample,), jnp.int32)
    f_out = jax.ShapeDtypeStruct((rows * nsample,), jnp.float32)
    scratch = [
        pltpu.VMEM((rpw,), jnp.float32),
        pltpu.VMEM((rpw,), jnp.float32),
        pltpu.VMEM((rpw,), jnp.float32),
        pltpu.VMEM((n,), jnp.float32),
        pltpu.VMEM((n,), jnp.float32),
        pltpu.VMEM((n,), jnp.float32),
    ]
    if coords:
        scratch += [pltpu.VMEM((n + 64,), jnp.float32)] * 12
        scratch += [pltpu.VMEM((rpw * nsample,), jnp.float32)] * 3
    else:
        scratch += [pltpu.VMEM((n + 64,), jnp.int32),
                    pltpu.VMEM((rpw * nsample,), jnp.int32)]

    def body(*refs):
        _select_body(refs, S=S, n=n, r2=r2, rpw=rpw, nsample=nsample,
                     coords=coords)

    kfn = pl.kernel(
        out_type=(f_out, f_out, f_out) if coords else i_out,
        mesh=_sc_mesh(),
        compiler_params=_sc_params(),
        scratch_types=scratch,
    )(body)
    out = kfn(cx.reshape(-1), cy.reshape(-1), cz.reshape(-1),
              X.reshape(-1), Y.reshape(-1), Z.reshape(-1))
    if coords:
        return out
    return out.reshape(rows, nsample)


def _gather_body(tab_hbm, idx_hbm, out_hbm, idxv, rows0, rows1,
                 sg0, sg1, so0, so1, *, tpw, chunk):
    w = lax.axis_index("s") * 2 + lax.axis_index("c")
    base = w * tpw
    nc = tpw // chunk
    rows = (rows0, rows1)
    sg = (sg0, sg1)
    so = (so0, so1)
    pltpu.sync_copy(idx_hbm.at[pl.ds(base, tpw)], idxv)
    gathers = [None, None]
    outs = [None, None]
    gathers[0] = pltpu.async_copy(
        tab_hbm.at[idxv.at[pl.ds(0, chunk)]], rows[0], sg[0])
    for c in range(nc):
        bi = c & 1
        gathers[bi].wait()
        if c + 1 < nc:
            ni = (c + 1) & 1
            if outs[ni] is not None:
                outs[ni].wait()
                outs[ni] = None
            gathers[ni] = pltpu.async_copy(
                tab_hbm.at[idxv.at[pl.ds((c + 1) * chunk, chunk)]],
                rows[ni], sg[ni])
        outs[bi] = pltpu.async_copy(
            rows[bi], out_hbm.at[pl.ds(base + c * chunk, chunk)], so[bi])
    for o in outs:
        if o is not None:
            o.wait()


def _gather(tab, idx, chunk):
    tot = idx.shape[0]
    d = tab.shape[1]
    tpw = tot // _NW
    kfn = pl.kernel(
        out_type=jax.ShapeDtypeStruct((tot, d), tab.dtype),
        mesh=_sc_mesh(),
        compiler_params=_sc_params(),
        scratch_types=[
            pltpu.VMEM((tpw,), jnp.int32),
            pltpu.VMEM((chunk, d), tab.dtype),
            pltpu.VMEM((chunk, d), tab.dtype),
            pltpu.SemaphoreType.DMA,
            pltpu.SemaphoreType.DMA,
            pltpu.SemaphoreType.DMA,
            pltpu.SemaphoreType.DMA,
        ],
    )(functools.partial(_gather_body, tpw=tpw, chunk=chunk))
    return kfn(tab, idx)


def _mlp1_body(xg_ref, yg_ref, zg_ref, ct_ref, w1_ref, b1_ref, w2_ref,
               b2_ref, w3_ref, b3_ref, w1a2_ref, w1b2_ref, u2_ref,
               *, st, ns):
    xg = xg_ref[0, 0]
    yg = yg_ref[0, 0]
    zg = zg_ref[0, 0]
    w1 = w1_ref[...]
    z1 = w1[:, 0:1] * xg + w1[:, 1:2] * yg + w1[:, 2:3] * zg + b1_ref[...]
    z1 = jnp.maximum(z1, 0.0)
    z2 = jnp.dot(w2_ref[...], z1, preferred_element_type=jnp.float32)
    z2 = jnp.maximum(z2 + b2_ref[...], 0.0)
    z3 = jnp.dot(w3_ref[...], z2, preferred_element_type=jnp.float32)
    z3 = jnp.maximum(z3 + b3_ref[...], 0.0)
    m = z3
    width = st * ns
    while width > st:
        width //= 2
        m = jnp.maximum(m[:, :width], m[:, width:2 * width])
    feat = m
    u2 = jnp.dot(w1a2_ref[...], ct_ref[0, 0], preferred_element_type=jnp.float32)
    u2 = u2 + jnp.dot(w1b2_ref[...], feat, preferred_element_type=jnp.float32)
    u2_ref[0, 0] = u2


def _mlp1(Xg, Yg, Zg, CT, w1, b1, w2, b2, w3, b3, w1a2, w1b2,
          st=128, ns=64):
    S = CT.shape[-1]
    nt = S // st
    o_next = w1a2.shape[0]
    Xr = Xg.reshape(B, nt, 1, st * ns)
    Yr = Yg.reshape(B, nt, 1, st * ns)
    Zr = Zg.reshape(B, nt, 1, st * ns)
    CTr = CT.reshape(B, 16, nt, st).transpose(0, 2, 1, 3)
    grid = (B, nt)
    def wspec(shape):
        return pl.BlockSpec(shape, lambda b_, t: tuple(0 for _ in shape))
    gspec = pl.BlockSpec((1, 1, 1, st * ns), lambda b_, t: (b_, t, 0, 0))
    out = pl.pallas_call(
        functools.partial(_mlp1_body, st=st, ns=ns),
        grid=grid,
        in_specs=[
            gspec, gspec, gspec,
            pl.BlockSpec((1, 1, 16, st), lambda b_, t: (b_, t, 0, 0)),
            wspec(w1.shape),
            wspec(b1.shape),
            wspec(w2.shape),
            wspec(b2.shape),
            wspec(w3.shape),
            wspec(b3.shape),
            wspec(w1a2.shape),
            wspec(w1b2.shape),
        ],
        out_specs=pl.BlockSpec((1, 1, o_next, st), lambda b_, t: (b_, t, 0, 0)),
        out_shape=jax.ShapeDtypeStruct((B, nt, o_next, st), jnp.float32),
    )(Xr, Yr, Zr, CTr, w1, b1, w2, b2, w3, b3, w1a2, w1b2)
    return out.transpose(0, 1, 3, 2).reshape(B * S, o_next)


def _tail_body(h_ref, w2t_ref, b2_ref, w3t_ref, b3_ref, fw1t_ref, fb1_ref,
               fw2t_ref, fb2_ref, fw3t_ref, fb3_ref, out_ref, *, npts):
    H = h_ref[...]
    z2 = jnp.dot(H, w2t_ref[...], preferred_element_type=jnp.float32)
    z2 = jnp.maximum(z2 + b2_ref[...], 0.0)
    z3 = jnp.dot(z2, w3t_ref[...], preferred_element_type=jnp.float32)
    z3 = jnp.maximum(z3 + b3_ref[...], 0.0)
    g = jnp.max(z3.reshape(B, npts, z3.shape[1]), axis=1)
    y = jnp.dot(g, fw1t_ref[...], preferred_element_type=jnp.float32)
    y = jnp.maximum(y + fb1_ref[...], 0.0)
    y = jnp.dot(y, fw2t_ref[...], preferred_element_type=jnp.float32)
    y = jnp.maximum(y + fb2_ref[...], 0.0)
    y = jnp.dot(y, fw3t_ref[...], preferred_element_type=jnp.float32)
    out_ref[...] = y + fb3_ref[...]


def _tail(H, w2t, b2, w3t, b3, fw1t, fb1, fw2t, fb2, fw3t, fb3):
    npts = H.shape[1]
    return pl.pallas_call(
        functools.partial(_tail_body, npts=npts),
        out_shape=jax.ShapeDtypeStruct((B, fw3t.shape[1]), jnp.float32),
    )(H.reshape(B * npts, H.shape[-1]), w2t, b2, w3t, b3,
      fw1t, fb1, fw2t, fb2, fw3t, fb3)


def _mlp2_body(g_ref, c_ref, w1a2t_ref, b1_ref, w2t_ref, b2_ref, w3t_ref,
               b3_ref, w1a3t_ref, w1b3t_ref, b1_3_ref, h_ref, *, st, ns):
    G = g_ref[0]
    C = c_ref[0]
    v = jnp.dot(C, w1a2t_ref[...], preferred_element_type=jnp.float32)
    c_in = G.shape[1]
    z1 = G.reshape(st, ns, c_in) - v[:, None, :] + b1_ref[...]
    z1 = jnp.maximum(z1, 0.0).reshape(st * ns, c_in)
    z2 = jnp.dot(z1, w2t_ref[...], preferred_element_type=jnp.float32)
    z2 = jnp.maximum(z2 + b2_ref[...], 0.0)
    z3 = jnp.dot(z2, w3t_ref[...], preferred_element_type=jnp.float32)
    o3 = z3.shape[1]
    z3 = jnp.maximum(z3 + b3_ref[...], 0.0)
    feat = jnp.max(z3.reshape(st, ns, o3), axis=1)
    h1 = jnp.dot(C, w1a3t_ref[...], preferred_element_type=jnp.float32)
    h1 = h1 + jnp.dot(feat, w1b3t_ref[...], preferred_element_type=jnp.float32)
    h_ref[0] = jnp.maximum(h1 + b1_3_ref[...], 0.0)


def _mlp2(G, C, w1a2t, b1, w2t, b2, w3t, b3, w1a3t, w1b3t, b1_3, ns=64):
    nb = C.shape[0]
    st = C.shape[1]
    o_out = w1a3t.shape[1]
    Gr = G.reshape(nb, st * ns, G.shape[-1])
    grid = (nb,)
    def wspec(shape):
        return pl.BlockSpec(shape, lambda b_: tuple(0 for _ in shape))
    return pl.pallas_call(
        functools.partial(_mlp2_body, st=st, ns=ns),
        grid=grid,
        in_specs=[
            pl.BlockSpec((1, st * ns, G.shape[-1]), lambda b_: (b_, 0, 0)),
            pl.BlockSpec((1, st, C.shape[-1]), lambda b_: (b_, 0, 0)),
            wspec(w1a2t.shape),
            wspec(b1.shape),
            wspec(w2t.shape),
            wspec(b2.shape),
            wspec(w3t.shape),
            wspec(b3.shape),
            wspec(w1a3t.shape),
            wspec(w1b3t.shape),
            wspec(b1_3.shape),
        ],
        out_specs=pl.BlockSpec((1, st, o_out), lambda b_: (b_, 0, 0)),
        out_shape=jax.ShapeDtypeStruct((nb, st, o_out), jnp.float32),
    )(Gr, C, w1a2t, b1, w2t, b2, w3t, b3, w1a3t, w1b3t, b1_3)


def _padT(w, rows=16):
    wt = w.T
    return jnp.pad(wt, ((0, rows - wt.shape[0]), (0, 0)))


def kernel(pointcloud, params):
    X = pointcloud[..., 0]
    Y = pointcloud[..., 1]
    Z = pointcloud[..., 2]
    p1, p2, p3, fc = params["sa1"], params["sa2"], params["sa3"], params["fc"]

    w1p = jnp.pad(_fold(p1["w"][0], p1["gamma"][0]),
                  ((0, 0), (0, 13)))
    b1c = p1["beta"][0].reshape(-1, 1)
    w2r = _fold(p1["w"][1], p1["gamma"][1])
    b2c = p1["beta"][1].reshape(-1, 1)
    w3r = _fold(p1["w"][2], p1["gamma"][2])
    b3c = p1["beta"][2].reshape(-1, 1)
    w1_2 = _fold(p2["w"][0], p2["gamma"][0])
    w1a2p = jnp.pad(w1_2[:, :3], ((0, 0), (0, 13)))
    w1a2t = _padT(w1_2[:, :3])
    w1b2r = w1_2[:, 3:]
    b1_2 = p2["beta"][0].reshape(1, -1)
    w2_2t = _fold(p2["w"][1], p2["gamma"][1]).T
    b2_2 = p2["beta"][1].reshape(1, -1)
    w3_2t = _fold(p2["w"][2], p2["gamma"][2]).T
    b3_2 = p2["beta"][2].reshape(1, -1)
    w1_3 = _fold(p3["w"][0], p3["gamma"][0])
    w1a3t = _padT(w1_3[:, :3])
    w1b3t = w1_3[:, 3:].T
    b1_3 = p3["beta"][0].reshape(1, -1)
    w2_3t = _fold(p3["w"][1], p3["gamma"][1]).T
    b2_3 = p3["beta"][1].reshape(1, -1)
    w3_3t = _fold(p3["w"][2], p3["gamma"][2]).T
    b3_3 = p3["beta"][2].reshape(1, -1)
    fw1t = _fold(fc["w1"], fc["g1"]).T
    fw2t = _fold(fc["w2"], fc["g2"]).T
    fw3t = fc["w3"].T

    cx1, cy1, cz1 = _fps(X, Y, Z, 512)
    Xg, Yg, Zg = _select(cx1, cy1, cz1, X, Y, Z, 0.2 * 0.2, coords=True)
    CT1 = jnp.pad(jnp.stack([cx1, cy1, cz1], axis=1), ((0, 0), (0, 13), (0, 0)))
    u2tab = _mlp1(Xg, Yg, Zg, CT1, w1p, b1c, w2r, b2c, w3r, b3c, w1a2p, w1b2r)

    cx2, cy2, cz2 = _fps(cx1, cy1, cz1, 128)
    idx2 = _select(cx2, cy2, cz2, cx1, cy1, cz1, 0.4 * 0.4)
    C2 = jnp.pad(jnp.stack([cx2, cy2, cz2], axis=-1), ((0, 0), (0, 0), (0, 13)))
    G2 = _gather(u2tab, idx2.reshape(-1), chunk=256)
    H = _mlp2(G2, C2, w1a2t, b1_2, w2_2t, b2_2, w3_2t, b3_2,
              w1a3t, w1b3t, b1_3)

    return _tail(H, w2_3t, b2_3, w3_3t, b3_3,
                 fw1t, fc["b1"].reshape(1, -1), fw2t, fc["b2"].reshape(1, -1),
                 fw3t, fc["bias3"].reshape(1, -1))


# speedup vs baseline: 1.0831x; 1.0781x over previous
import functools

import jax
import jax.numpy as jnp
import numpy as np
from jax import lax
from jax.experimental import pallas as pl
from jax.experimental.pallas import tpu as pltpu
from jax.experimental.pallas import tpu_sc as plsc

B = 16
N0 = 1024
EPS = 1e-5
_SCALE = float(1.0 / np.sqrt(1.0 + EPS))

_NW = 32


@functools.lru_cache(maxsize=1)
def _sc_mesh():
    return plsc.VectorSubcoreMesh(core_axis_name="c", subcore_axis_name="s")


@functools.lru_cache(maxsize=1)
def _sc_params():
    return pltpu.CompilerParams(needs_layout_passes=False,
                                use_tc_tiling_on_sc=False)


def _fold(w, g):
    return w * (g * _SCALE)[:, None]


def _fps_body(x_ref, y_ref, z_ref, ocx_ref, ocy_ref, ocz_ref, *, npoint, n):
    X = x_ref[...]
    Y = y_ref[...]
    Z = z_ref[...]
    iota = lax.broadcasted_iota(jnp.int32, (B, n), 1)
    piota = lax.broadcasted_iota(jnp.int32, (B, npoint), 1)

    def body(i, st):
        dists, far, ocx, ocy, ocz = st
        oh = jnp.where(iota == far, 1.0, 0.0)
        cx = jnp.sum(X * oh, axis=1, keepdims=True)
        cy = jnp.sum(Y * oh, axis=1, keepdims=True)
        cz = jnp.sum(Z * oh, axis=1, keepdims=True)
        sel = piota == i
        ocx = jnp.where(sel, cx, ocx)
        ocy = jnp.where(sel, cy, ocy)
        ocz = jnp.where(sel, cz, ocz)
        dx = X - cx
        dy = Y - cy
        dz = Z - cz
        d = (dx * dx + dy * dy) + dz * dz
        dists = jnp.minimum(dists, d)
        far = jnp.argmax(dists, axis=1, keepdims=True).astype(jnp.int32)
        return dists, far, ocx, ocy, ocz

    dists = jnp.full((B, n), 1e10, jnp.float32)
    far = jnp.zeros((B, 1), jnp.int32)
    zc = jnp.zeros((B, npoint), jnp.float32)
    _, _, ocx, ocy, ocz = lax.fori_loop(0, npoint, body, (dists, far, zc, zc, zc))
    ocx_ref[...] = ocx
    ocy_ref[...] = ocy
    ocz_ref[...] = ocz


def _fps(X, Y, Z, npoint):
    n = X.shape[1]
    out = jax.ShapeDtypeStruct((B, npoint), jnp.float32)
    return pl.pallas_call(
        functools.partial(_fps_body, npoint=npoint, n=n),
        out_shape=(out, out, out),
    )(X, Y, Z)


def _select_body(refs, *, S, n, r2, rpw, nsample, coords):
    if coords:
        (cx_hbm, cy_hbm, cz_hbm, x_hbm, y_hbm, z_hbm,
         ox_hbm, oy_hbm, oz_hbm,
         cxv, cyv, czv, xv, yv, zv,
         xb0, yb0, zb0, xb1, yb1, zb1, xb2, yb2, zb2, xb3, yb3, zb3,
         oxv, oyv, ozv) = refs
        xbs = (xb0, xb1, xb2, xb3)
        ybs = (yb0, yb1, yb2, yb3)
        zbs = (zb0, zb1, zb2, zb3)
    else:
        (cx_hbm, cy_hbm, cz_hbm, x_hbm, y_hbm, z_hbm,
         oi_hbm, cxv, cyv, czv, xv, yv, zv, bufv, oiv) = refs
    w = lax.axis_index("s") * 2 + lax.axis_index("c")
    base_row = w * rpw
    b = base_row // S
    base_n = b * n
    pltpu.sync_copy(cx_hbm.at[pl.ds(base_row, rpw)], cxv)
    pltpu.sync_copy(cy_hbm.at[pl.ds(base_row, rpw)], cyv)
    pltpu.sync_copy(cz_hbm.at[pl.ds(base_row, rpw)], czv)
    pltpu.sync_copy(x_hbm.at[pl.ds(base_n, n)], xv)
    pltpu.sync_copy(y_hbm.at[pl.ds(base_n, n)], yv)
    pltpu.sync_copy(z_hbm.at[pl.ds(base_n, n)], zv)
    io16 = lax.iota(jnp.int32, 16)

    def quad_fn(q, _):
        r0 = q * 4
        cs = []
        for i in range(4):
            rvec = jnp.broadcast_to(r0 + i, (16,))
            cs.append((plsc.load_gather(cxv, [rvec]),
                       plsc.load_gather(cyv, [rvec]),
                       plsc.load_gather(czv, [rvec])))
        x0 = jnp.broadcast_to(xv[pl.ds(0, 16)][0], (16,))
        y0 = jnp.broadcast_to(yv[pl.ds(0, 16)][0], (16,))
        z0 = jnp.broadcast_to(zv[pl.ds(0, 16)][0], (16,))
        for i in range(4):
            xbs[i][pl.ds(0, 16)] = x0 - cs[i][0]
            ybs[i][pl.ds(0, 16)] = y0 - cs[i][1]
            zbs[i][pl.ds(0, 16)] = z0 - cs[i][2]

        def chunk_fn(j, cnts):
            off = j * 16
            xc = xv[pl.ds(off, 16)]
            yc = yv[pl.ds(off, 16)]
            zc = zv[pl.ds(off, 16)]
            out = []
            for i in range(4):
                dx = xc - cs[i][0]
                dy = yc - cs[i][1]
                dz = zc - cs[i][2]
                d = (dx * dx + dy * dy) + dz * dz
                m = d <= r2
                plsc.store_compressed(xbs[i].at[pl.ds(cnts[i], 16)], dx, mask=m)
                plsc.store_compressed(ybs[i].at[pl.ds(cnts[i], 16)], dy, mask=m)
                plsc.store_compressed(zbs[i].at[pl.ds(cnts[i], 16)], dz, mask=m)
                out.append(cnts[i] + plsc.all_reduce_population_count(m)[0])
            return tuple(out)

        cnts = lax.fori_loop(0, n // 16, chunk_fn,
                             (jnp.int32(0),) * 4, unroll=2)
        for i in range(4):
            r = r0 + i
            cnt = cnts[i]
            fx = jnp.broadcast_to(xbs[i][pl.ds(0, 16)][0], (16,))
            fy = jnp.broadcast_to(ybs[i][pl.ds(0, 16)][0], (16,))
            fz = jnp.broadcast_to(zbs[i][pl.ds(0, 16)][0], (16,))
            for kc in range(nsample // 16):
                xbs[i][pl.ds(cnt + kc * 16, 16)] = fx
                ybs[i][pl.ds(cnt + kc * 16, 16)] = fy
                zbs[i][pl.ds(cnt + kc * 16, 16)] = fz
            tile = r // 128
            s_in = r - tile * 128
            obase = tile * (128 * nsample) + s_in
            for kc in range(nsample // 16):
                oidx = (io16 + kc * 16) * 128 + obase
                plsc.store_scatter(oxv, [oidx], xbs[i][pl.ds(kc * 16, 16)])
                plsc.store_scatter(oyv, [oidx], ybs[i][pl.ds(kc * 16, 16)])
                plsc.store_scatter(ozv, [oidx], zbs[i][pl.ds(kc * 16, 16)])
        return 0

    def row_fn(r, _):
        rvec = jnp.broadcast_to(r, (16,))
        cxr = plsc.load_gather(cxv, [rvec])
        cyr = plsc.load_gather(cyv, [rvec])
        czr = plsc.load_gather(czv, [rvec])
        bufv[pl.ds(0, 16)] = jnp.broadcast_to(base_n, (16,))

        def chunk_fn(j, cnt):
            off = j * 16
            xc = xv[pl.ds(off, 16)]
            yc = yv[pl.ds(off, 16)]
            zc = zv[pl.ds(off, 16)]
            dx = xc - cxr
            dy = yc - cyr
            dz = zc - czr
            d = (dx * dx + dy * dy) + dz * dz
            m = d <= r2
            gidx = io16 + (off + base_n)
            plsc.store_compressed(bufv.at[pl.ds(cnt, 16)], gidx, mask=m)
            return cnt + plsc.all_reduce_population_count(m)[0]

        cnt = lax.fori_loop(0, n // 16, chunk_fn, jnp.int32(0), unroll=4)
        fill = jnp.broadcast_to(bufv[pl.ds(0, 16)][0], (16,))
        for kc in range(nsample // 16):
            bufv[pl.ds(cnt + kc * 16, 16)] = fill
        for kc in range(nsample // 16):
            oiv[pl.ds(r * nsample + kc * 16, 16)] = bufv[pl.ds(kc * 16, 16)]
        return 0

    if coords:
        lax.fori_loop(0, rpw // 4, quad_fn, 0)
    else:
        lax.fori_loop(0, rpw, row_fn, 0)
    sl = pl.ds(base_row * nsample, rpw * nsample)
    if coords:
        pltpu.sync_copy(oxv, ox_hbm.at[sl])
        pltpu.sync_copy(oyv, oy_hbm.at[sl])
        pltpu.sync_copy(ozv, oz_hbm.at[sl])
    else:
        pltpu.sync_copy(oiv, oi_hbm.at[sl])


def _select(cx, cy, cz, X, Y, Z, r2, nsample=64, coords=False):
    S = cx.shape[1]
    n = X.shape[1]
    rows = B * S
    rpw = rows // _NW
    i_out = jax.ShapeDtypeStruct((rows * nsample,), jnp.int32)
    f_out = jax.ShapeDtypeStruct((rows * nsample,), jnp.float32)
    scratch = [
        pltpu.VMEM((rpw,), jnp.float32),
        pltpu.VMEM((rpw,), jnp.float32),
        pltpu.VMEM((rpw,), jnp.float32),
        pltpu.VMEM((n,), jnp.float32),
        pltpu.VMEM((n,), jnp.float32),
        pltpu.VMEM((n,), jnp.float32),
    ]
    if coords:
        scratch += [pltpu.VMEM((n + 64,), jnp.float32)] * 12
        scratch += [pltpu.VMEM((rpw * nsample,), jnp.float32)] * 3
    else:
        scratch += [pltpu.VMEM((n + 64,), jnp.int32),
                    pltpu.VMEM((rpw * nsample,), jnp.int32)]

    def body(*refs):
        _select_body(refs, S=S, n=n, r2=r2, rpw=rpw, nsample=nsample,
                     coords=coords)

    kfn = pl.kernel(
        out_type=(f_out, f_out, f_out) if coords else i_out,
        mesh=_sc_mesh(),
        compiler_params=_sc_params(),
        scratch_types=scratch,
    )(body)
    out = kfn(cx.reshape(-1), cy.reshape(-1), cz.reshape(-1),
              X.reshape(-1), Y.reshape(-1), Z.reshape(-1))
    if coords:
        return out
    return out.reshape(rows, nsample)


def _gather_body(tab_hbm, idx_hbm, out_hbm, idxv, rows0, rows1,
                 sg0, sg1, so0, so1, *, tpw, chunk):
    w = lax.axis_index("s") * 2 + lax.axis_index("c")
    base = w * tpw
    nc = tpw // chunk
    rows = (rows0, rows1)
    sg = (sg0, sg1)
    so = (so0, so1)
    pltpu.sync_copy(idx_hbm.at[pl.ds(base, tpw)], idxv)
    gathers = [None, None]
    outs = [None, None]
    gathers[0] = pltpu.async_copy(
        tab_hbm.at[idxv.at[pl.ds(0, chunk)]], rows[0], sg[0])
    for c in range(nc):
        bi = c & 1
        gathers[bi].wait()
        if c + 1 < nc:
            ni = (c + 1) & 1
            if outs[ni] is not None:
                outs[ni].wait()
                outs[ni] = None
            gathers[ni] = pltpu.async_copy(
                tab_hbm.at[idxv.at[pl.ds((c + 1) * chunk, chunk)]],
                rows[ni], sg[ni])
        outs[bi] = pltpu.async_copy(
            rows[bi], out_hbm.at[pl.ds(base + c * chunk, chunk)], so[bi])
    for o in outs:
        if o is not None:
            o.wait()


def _gather(tab, idx, chunk):
    tot = idx.shape[0]
    d = tab.shape[1]
    tpw = tot // _NW
    kfn = pl.kernel(
        out_type=jax.ShapeDtypeStruct((tot, d), tab.dtype),
        mesh=_sc_mesh(),
        compiler_params=_sc_params(),
        scratch_types=[
            pltpu.VMEM((tpw,), jnp.int32),
            pltpu.VMEM((chunk, d), tab.dtype),
            pltpu.VMEM((chunk, d), tab.dtype),
            pltpu.SemaphoreType.DMA,
            pltpu.SemaphoreType.DMA,
            pltpu.SemaphoreType.DMA,
            pltpu.SemaphoreType.DMA,
        ],
    )(functools.partial(_gather_body, tpw=tpw, chunk=chunk))
    return kfn(tab, idx)


def _mlp1_body(xg_ref, yg_ref, zg_ref, ct_ref, w1_ref, b1_ref, w2_ref,
               b2_ref, w3_ref, b3_ref, w1a2_ref, w1b2_ref, u2_ref,
               *, st, ns):
    xg = xg_ref[0, 0]
    yg = yg_ref[0, 0]
    zg = zg_ref[0, 0]
    w1 = w1_ref[...]
    z1 = w1[:, 0:1] * xg + w1[:, 1:2] * yg + w1[:, 2:3] * zg + b1_ref[...]
    z1 = jnp.maximum(z1, 0.0)
    z2 = jnp.dot(w2_ref[...], z1, preferred_element_type=jnp.float32)
    z2 = jnp.maximum(z2 + b2_ref[...], 0.0)
    z3 = jnp.dot(w3_ref[...], z2, preferred_element_type=jnp.float32)
    z3 = jnp.maximum(z3 + b3_ref[...], 0.0)
    m = z3
    width = st * ns
    while width > st:
        width //= 2
        m = jnp.maximum(m[:, :width], m[:, width:2 * width])
    feat = m
    u2 = jnp.dot(w1a2_ref[...], ct_ref[0, 0], preferred_element_type=jnp.float32)
    u2 = u2 + jnp.dot(w1b2_ref[...], feat, preferred_element_type=jnp.float32)
    u2_ref[0, 0] = u2


def _mlp1(Xg, Yg, Zg, CT, w1, b1, w2, b2, w3, b3, w1a2, w1b2,
          st=128, ns=64):
    S = CT.shape[-1]
    nt = S // st
    o_next = w1a2.shape[0]
    Xr = Xg.reshape(B, nt, 1, st * ns)
    Yr = Yg.reshape(B, nt, 1, st * ns)
    Zr = Zg.reshape(B, nt, 1, st * ns)
    CTr = CT.reshape(B, 16, nt, st).transpose(0, 2, 1, 3)
    grid = (B, nt)
    def wspec(shape):
        return pl.BlockSpec(shape, lambda b_, t: tuple(0 for _ in shape))
    gspec = pl.BlockSpec((1, 1, 1, st * ns), lambda b_, t: (b_, t, 0, 0))
    out = pl.pallas_call(
        functools.partial(_mlp1_body, st=st, ns=ns),
        grid=grid,
        in_specs=[
            gspec, gspec, gspec,
            pl.BlockSpec((1, 1, 16, st), lambda b_, t: (b_, t, 0, 0)),
            wspec(w1.shape),
            wspec(b1.shape),
            wspec(w2.shape),
            wspec(b2.shape),
            wspec(w3.shape),
            wspec(b3.shape),
            wspec(w1a2.shape),
            wspec(w1b2.shape),
        ],
        out_specs=pl.BlockSpec((1, 1, o_next, st), lambda b_, t: (b_, t, 0, 0)),
        out_shape=jax.ShapeDtypeStruct((B, nt, o_next, st), jnp.float32),
    )(Xr, Yr, Zr, CTr, w1, b1, w2, b2, w3, b3, w1a2, w1b2)
    return out.transpose(0, 1, 3, 2).reshape(B * S, o_next)


def _tail_body(h_ref, w2t_ref, b2_ref, w3t_ref, b3_ref, fw1t_ref, fb1_ref,
               fw2t_ref, fb2_ref, fw3t_ref, fb3_ref, out_ref, *, npts):
    H = h_ref[...]
    z2 = jnp.dot(H, w2t_ref[...], preferred_element_type=jnp.float32)
    z2 = jnp.maximum(z2 + b2_ref[...], 0.0)
    z3 = jnp.dot(z2, w3t_ref[...], preferred_element_type=jnp.float32)
    z3 = jnp.maximum(z3 + b3_ref[...], 0.0)
    g = jnp.max(z3.reshape(B, npts, z3.shape[1]), axis=1)
    y = jnp.dot(g, fw1t_ref[...], preferred_element_type=jnp.float32)
    y = jnp.maximum(y + fb1_ref[...], 0.0)
    y = jnp.dot(y, fw2t_ref[...], preferred_element_type=jnp.float32)
    y = jnp.maximum(y + fb2_ref[...], 0.0)
    y = jnp.dot(y, fw3t_ref[...], preferred_element_type=jnp.float32)
    out_ref[...] = y + fb3_ref[...]


def _tail(H, w2t, b2, w3t, b3, fw1t, fb1, fw2t, fb2, fw3t, fb3):
    npts = H.shape[1]
    return pl.pallas_call(
        functools.partial(_tail_body, npts=npts),
        out_shape=jax.ShapeDtypeStruct((B, fw3t.shape[1]), jnp.float32),
    )(H.reshape(B * npts, H.shape[-1]), w2t, b2, w3t, b3,
      fw1t, fb1, fw2t, fb2, fw3t, fb3)


def _mlp2_body(g_ref, c_ref, w1a2t_ref, b1_ref, w2t_ref, b2_ref, w3t_ref,
               b3_ref, w1a3t_ref, w1b3t_ref, b1_3_ref, h_ref, *, st, ns):
    G = g_ref[0]
    C = c_ref[0]
    v = jnp.dot(C, w1a2t_ref[...], preferred_element_type=jnp.float32)
    c_in = G.shape[1]
    z1 = G.reshape(st, ns, c_in) - v[:, None, :] + b1_ref[...]
    z1 = jnp.maximum(z1, 0.0).reshape(st * ns, c_in)
    z2 = jnp.dot(z1, w2t_ref[...], preferred_element_type=jnp.float32)
    z2 = jnp.maximum(z2 + b2_ref[...], 0.0)
    z3 = jnp.dot(z2, w3t_ref[...], preferred_element_type=jnp.float32)
    o3 = z3.shape[1]
    z3 = jnp.maximum(z3 + b3_ref[...], 0.0)
    feat = jnp.max(z3.reshape(st, ns, o3), axis=1)
    h1 = jnp.dot(C, w1a3t_ref[...], preferred_element_type=jnp.float32)
    h1 = h1 + jnp.dot(feat, w1b3t_ref[...], preferred_element_type=jnp.float32)
    h_ref[0] = jnp.maximum(h1 + b1_3_ref[...], 0.0)


def _mlp2(G, C, w1a2t, b1, w2t, b2, w3t, b3, w1a3t, w1b3t, b1_3, ns=64):
    nb = C.shape[0]
    st = C.shape[1]
    o_out = w1a3t.shape[1]
    Gr = G.reshape(nb, st * ns, G.shape[-1])
    grid = (nb,)
    def wspec(shape):
        return pl.BlockSpec(shape, lambda b_: tuple(0 for _ in shape))
    return pl.pallas_call(
        functools.partial(_mlp2_body, st=st, ns=ns),
        grid=grid,
        in_specs=[
            pl.BlockSpec((1, st * ns, G.shape[-1]), lambda b_: (b_, 0, 0)),
            pl.BlockSpec((1, st, C.shape[-1]), lambda b_: (b_, 0, 0)),
            wspec(w1a2t.shape),
            wspec(b1.shape),
            wspec(w2t.shape),
            wspec(b2.shape),
            wspec(w3t.shape),
            wspec(b3.shape),
            wspec(w1a3t.shape),
            wspec(w1b3t.shape),
            wspec(b1_3.shape),
        ],
        out_specs=pl.BlockSpec((1, st, o_out), lambda b_: (b_, 0, 0)),
        out_shape=jax.ShapeDtypeStruct((nb, st, o_out), jnp.float32),
    )(Gr, C, w1a2t, b1, w2t, b2, w3t, b3, w1a3t, w1b3t, b1_3)


def _padT(w, rows=16):
    wt = w.T
    return jnp.pad(wt, ((0, rows - wt.shape[0]), (0, 0)))


def kernel(pointcloud, params):
    X = pointcloud[..., 0]
    Y = pointcloud[..., 1]
    Z = pointcloud[..., 2]
    p1, p2, p3, fc = params["sa1"], params["sa2"], params["sa3"], params["fc"]

    w1p = jnp.pad(_fold(p1["w"][0], p1["gamma"][0]),
                  ((0, 0), (0, 13)))
    b1c = p1["beta"][0].reshape(-1, 1)
    w2r = _fold(p1["w"][1], p1["gamma"][1])
    b2c = p1["beta"][1].reshape(-1, 1)
    w3r = _fold(p1["w"][2], p1["gamma"][2])
    b3c = p1["beta"][2].reshape(-1, 1)
    w1_2 = _fold(p2["w"][0], p2["gamma"][0])
    w1a2p = jnp.pad(w1_2[:, :3], ((0, 0), (0, 13)))
    w1a2t = _padT(w1_2[:, :3])
    w1b2r = w1_2[:, 3:]
    b1_2 = p2["beta"][0].reshape(1, -1)
    w2_2t = _fold(p2["w"][1], p2["gamma"][1]).T
    b2_2 = p2["beta"][1].reshape(1, -1)
    w3_2t = _fold(p2["w"][2], p2["gamma"][2]).T
    b3_2 = p2["beta"][2].reshape(1, -1)
    w1_3 = _fold(p3["w"][0], p3["gamma"][0])
    w1a3t = _padT(w1_3[:, :3])
    w1b3t = w1_3[:, 3:].T
    b1_3 = p3["beta"][0].reshape(1, -1)
    w2_3t = _fold(p3["w"][1], p3["gamma"][1]).T
    b2_3 = p3["beta"][1].reshape(1, -1)
    w3_3t = _fold(p3["w"][2], p3["gamma"][2]).T
    b3_3 = p3["beta"][2].reshape(1, -1)
    fw1t = _fold(fc["w1"], fc["g1"]).T
    fw2t = _fold(fc["w2"], fc["g2"]).T
    fw3t = fc["w3"].T

    cx1, cy1, cz1 = _fps(X, Y, Z, 512)
    Xg, Yg, Zg = _select(cx1, cy1, cz1, X, Y, Z, 0.2 * 0.2, coords=True)
    CT1 = jnp.pad(jnp.stack([cx1, cy1, cz1], axis=1), ((0, 0), (0, 13), (0, 0)))
    u2tab = _mlp1(Xg, Yg, Zg, CT1, w1p, b1c, w2r, b2c, w3r, b3c, w1a2p, w1b2r)

    cx2, cy2, cz2 = _fps(cx1, cy1, cz1, 128)
    idx2 = _select(cx2, cy2, cz2, cx1, cy1, cz1, 0.4 * 0.4)
    C2 = jnp.pad(jnp.stack([cx2, cy2, cz2], axis=-1), ((0, 0), (0, 0), (0, 13)))
    G2 = _gather(u2tab, idx2.reshape(-1), chunk=256)
    H = _mlp2(G2, C2, w1a2t, b1_2, w2_2t, b2_2, w3_2t, b3_2,
              w1a3t, w1b3t, b1_3)

    return _tail(H, w2_3t, b2_3, w3_3t, b3_3,
                 fw1t, fc["b1"].reshape(1, -1), fw2t, fc["b2"].reshape(1, -1),
                 fw3t, fc["bias3"].reshape(1, -1))
